# Initial kernel scaffold; baseline (speedup 1.0000x reference)
import functools

import jax
import jax.numpy as jnp
from jax import lax
from jax.experimental import pallas as pl
from jax.experimental.pallas import tpu as pltpu
from jax.experimental.pallas import tpu_sc as plsc

NC = 2
NS = 16
NW = NC * NS
CH = 128
NBUF = 4


def _make_gather(n_rows: int, d: int):
    assert n_rows % (NW * CH) == 0
    per_w = n_rows // NW
    n_ch = per_w // CH
    assert n_ch % NBUF == 0 and n_ch >= 2 * NBUF

    mesh = plsc.VectorSubcoreMesh(core_axis_name="c", subcore_axis_name="s")

    @functools.partial(
        pl.kernel,
        mesh=mesh,
        out_type=jax.ShapeDtypeStruct((n_rows, d), jnp.float32),
        scratch_types=[
            pltpu.VMEM((per_w,), jnp.int32),
            pltpu.VMEM((NBUF, CH, d), jnp.float32),
            pltpu.SemaphoreType.DMA,
            pltpu.SemaphoreType.DMA,
            pltpu.SemaphoreType.DMA,
            pltpu.SemaphoreType.DMA,
            pltpu.SemaphoreType.DMA,
            pltpu.SemaphoreType.DMA,
            pltpu.SemaphoreType.DMA,
            pltpu.SemaphoreType.DMA,
        ],
    )
    def gather_kernel(idx_hbm, table_hbm, out_hbm, idx_v, rows_v,
                      gs0, gs1, gs2, gs3, os0, os1, os2, os3):
        gsems = [gs0, gs1, gs2, gs3]
        osems = [os0, os1, os2, os3]
        wid = lax.axis_index("c") * NS + lax.axis_index("s")
        base = wid * per_w

        pltpu.sync_copy(idx_hbm.at[pl.ds(base, per_w)], idx_v)

        def gather_start(g, b):
            pltpu.async_copy(
                table_hbm.at[idx_v.at[pl.ds(g * CH, CH)]], rows_v.at[b],
                gsems[b])

        def gather_wait(g, b):
            pltpu.make_async_copy(
                table_hbm.at[idx_v.at[pl.ds(g * CH, CH)]], rows_v.at[b],
                gsems[b]).wait()

        def out_start(g, b):
            pltpu.async_copy(
                rows_v.at[b], out_hbm.at[pl.ds(base + g * CH, CH)], osems[b])

        def out_wait(g, b):
            pltpu.make_async_copy(
                rows_v.at[b], out_hbm.at[pl.ds(base + g * CH, CH)],
                osems[b]).wait()

        gather_start(0, 0)
        gather_start(1, 1)
        gather_wait(0, 0)
        out_start(0, 0)
        gather_start(2, 2)
        gather_wait(1, 1)
        out_start(1, 1)
        gather_start(3, 3)
        gather_wait(2, 2)
        out_start(2, 2)
        out_wait(0, 0)
        gather_start(4, 0)
        gather_wait(3, 3)
        out_start(3, 3)
        out_wait(1, 1)
        gather_start(5, 1)

        def group(gidx, carry):
            g0 = gidx * NBUF
            for b in range(NBUF):
                g = g0 + b
                b2 = (b + 2) % NBUF
                gather_wait(g, b)
                out_start(g, b)
                out_wait(g - 2, b2)
                gather_start(g + 2, b2)
            return carry

        lax.fori_loop(1, n_ch // NBUF - 1, group, 0)

        gl = n_ch - NBUF
        gather_wait(gl + 0, 0)
        out_start(gl + 0, 0)
        out_wait(gl - 2, 2)
        gather_start(gl + 2, 2)
        gather_wait(gl + 1, 1)
        out_start(gl + 1, 1)
        out_wait(gl - 1, 3)
        gather_start(gl + 3, 3)
        gather_wait(gl + 2, 2)
        out_start(gl + 2, 2)
        out_wait(gl + 0, 0)
        gather_wait(gl + 3, 3)
        out_start(gl + 3, 3)
        out_wait(gl + 1, 1)
        out_wait(gl + 2, 2)
        out_wait(gl + 3, 3)

    return gather_kernel


def kernel(q, table):
    b, h = q.shape
    v, d = table.shape
    idx = q.reshape(b * h).astype(jnp.int32)
    out = _make_gather(b * h, d)(idx, table)
    return out.reshape(b, h, d)


# speedup vs baseline: 1.8639x; 1.8639x over previous
import functools

import jax
import jax.numpy as jnp
from jax import lax
from jax.experimental import pallas as pl
from jax.experimental.pallas import tpu as pltpu
from jax.experimental.pallas import tpu_sc as plsc

NC = 2
NS = 16
NW = NC * NS
CH = 128
NBUF = 4


def _make_gather(n_rows: int, d: int):
    assert n_rows % (NW * CH) == 0
    per_w = n_rows // NW
    n_ch = per_w // CH
    assert n_ch % NBUF == 0 and n_ch >= 2 * NBUF

    mesh = plsc.VectorSubcoreMesh(core_axis_name="c", subcore_axis_name="s")

    @functools.partial(
        pl.kernel,
        mesh=mesh,
        compiler_params=pltpu.CompilerParams(use_tc_tiling_on_sc=False),
        out_type=jax.ShapeDtypeStruct((n_rows, d), jnp.float32),
        scratch_types=[
            pltpu.VMEM((per_w,), jnp.int32),
            pltpu.VMEM((NBUF, CH, d), jnp.float32),
            pltpu.SemaphoreType.DMA,
            pltpu.SemaphoreType.DMA,
            pltpu.SemaphoreType.DMA,
            pltpu.SemaphoreType.DMA,
            pltpu.SemaphoreType.DMA,
            pltpu.SemaphoreType.DMA,
            pltpu.SemaphoreType.DMA,
            pltpu.SemaphoreType.DMA,
        ],
    )
    def gather_kernel(idx_hbm, table_hbm, out_hbm, idx_v, rows_v,
                      gs0, gs1, gs2, gs3, os0, os1, os2, os3):
        gsems = [gs0, gs1, gs2, gs3]
        osems = [os0, os1, os2, os3]
        wid = lax.axis_index("c") * NS + lax.axis_index("s")
        base = wid * per_w

        pltpu.sync_copy(idx_hbm.at[pl.ds(base, per_w)], idx_v)

        def gather_start(g, b):
            pltpu.async_copy(
                table_hbm.at[idx_v.at[pl.ds(g * CH, CH)]], rows_v.at[b],
                gsems[b])

        def gather_wait(g, b):
            pltpu.make_async_copy(
                table_hbm.at[idx_v.at[pl.ds(g * CH, CH)]], rows_v.at[b],
                gsems[b]).wait()

        def out_start(g, b):
            pltpu.async_copy(
                rows_v.at[b], out_hbm.at[pl.ds(base + g * CH, CH)], osems[b])

        def out_wait(g, b):
            pltpu.make_async_copy(
                rows_v.at[b], out_hbm.at[pl.ds(base + g * CH, CH)],
                osems[b]).wait()

        gather_start(0, 0)
        gather_start(1, 1)
        gather_wait(0, 0)
        out_start(0, 0)
        gather_start(2, 2)
        gather_wait(1, 1)
        out_start(1, 1)
        gather_start(3, 3)
        gather_wait(2, 2)
        out_start(2, 2)
        out_wait(0, 0)
        gather_start(4, 0)
        gather_wait(3, 3)
        out_start(3, 3)
        out_wait(1, 1)
        gather_start(5, 1)

        def group(gidx, carry):
            g0 = gidx * NBUF
            for b in range(NBUF):
                g = g0 + b
                b2 = (b + 2) % NBUF
                gather_wait(g, b)
                out_start(g, b)
                out_wait(g - 2, b2)
                gather_start(g + 2, b2)
            return carry

        lax.fori_loop(1, n_ch // NBUF - 1, group, 0)

        gl = n_ch - NBUF
        gather_wait(gl + 0, 0)
        out_start(gl + 0, 0)
        out_wait(gl - 2, 2)
        gather_start(gl + 2, 2)
        gather_wait(gl + 1, 1)
        out_start(gl + 1, 1)
        out_wait(gl - 1, 3)
        gather_start(gl + 3, 3)
        gather_wait(gl + 2, 2)
        out_start(gl + 2, 2)
        out_wait(gl + 0, 0)
        gather_wait(gl + 3, 3)
        out_start(gl + 3, 3)
        out_wait(gl + 1, 1)
        out_wait(gl + 2, 2)
        out_wait(gl + 3, 3)

    return gather_kernel


def kernel(q, table):
    b, h = q.shape
    v, d = table.shape
    idx = q.reshape(b * h).astype(jnp.int32)
    out = _make_gather(b * h, d)(idx, table)
    return out.reshape(b, h, d)
